# baseline (speedup 1.0000x reference)
import jax
import jax.numpy as jnp
from jax.experimental import pallas as pl
from jax.experimental.pallas import tpu as pltpu

N = 10000
NUM_SYMPS = 360
ROW_BLOCK = 400
NUM_SPLITS = 2
SUB_ROWS = ROW_BLOCK // NUM_SPLITS


def _dot(a, b):
    return jnp.dot(a, b, precision=jax.lax.Precision.DEFAULT,
                   preferred_element_type=jnp.float32)


def _s1_kernel(x_ref, w1_ref, s1_ref):
    s1_ref[...] = _dot(x_ref[...], w1_ref[...])


def _pass1_kernel(s1_ref, b1_ref, w2_ref, *refs):
    adj_refs, s2_ref = refs[:NUM_SPLITS], refs[NUM_SPLITS]
    for j in range(NUM_SPLITS):
        h = jnp.maximum(_dot(adj_refs[j][...], s1_ref[...]) + b1_ref[...],
                        0.0)
        s2_ref[j * SUB_ROWS:(j + 1) * SUB_ROWS, :] = _dot(h, w2_ref[...])


def _pass2_kernel(s2_ref, b2_ref, wsh_t_ref, bsh_ref, whc_t_ref, bhc_ref,
                  *refs):
    adj_refs, out_ref = refs[:NUM_SPLITS], refs[NUM_SPLITS]
    base = pl.program_id(0) * ROW_BLOCK
    for j in range(NUM_SPLITS):
        h2 = _dot(adj_refs[j][...], s2_ref[...]) + b2_ref[...]
        logits_s = _dot(h2, wsh_t_ref[...]) + bsh_ref[...]
        logits_h = _dot(h2, whc_t_ref[...]) + bhc_ref[...]
        rows = (base + j * SUB_ROWS
                + jax.lax.broadcasted_iota(jnp.int32, (SUB_ROWS, 1), 0))
        out_ref[j * SUB_ROWS:(j + 1) * SUB_ROWS, :] = jax.nn.sigmoid(
            jnp.where(rows < NUM_SYMPS, logits_s, logits_h))


@jax.jit
def kernel(x, adj, W1, b1, W2, b2, Wsh, bsh, Whc, bhc):
    nfeat = x.shape[1]
    nhid = W1.shape[1]
    dim = W2.shape[1]
    nherbs = Wsh.shape[0]
    num_blocks = N // ROW_BLOCK

    s1 = pl.pallas_call(
        _s1_kernel,
        out_shape=jax.ShapeDtypeStruct((N, nhid), jnp.float32),
    )(x, W1)

    full = lambda shape: pl.BlockSpec(shape, lambda i: (0, 0))

    def adj_spec(j):
        return pl.BlockSpec((SUB_ROWS, N),
                            lambda i, j=j: (NUM_SPLITS * i + j, 0))

    adj_specs = [adj_spec(j) for j in range(NUM_SPLITS)]
    adj_args = [adj] * NUM_SPLITS

    s2 = pl.pallas_call(
        _pass1_kernel,
        grid=(num_blocks,),
        in_specs=[
            full((N, nhid)),
            full((1, nhid)),
            full((nhid, dim)),
            *adj_specs,
        ],
        out_specs=pl.BlockSpec((ROW_BLOCK, dim), lambda i: (i, 0)),
        out_shape=jax.ShapeDtypeStruct((N, dim), jnp.float32),
        compiler_params=pltpu.CompilerParams(
            dimension_semantics=("arbitrary",)),
    )(s1, b1.reshape(1, nhid), W2, *adj_args)

    out = pl.pallas_call(
        _pass2_kernel,
        grid=(num_blocks,),
        in_specs=[
            full((N, dim)),
            full((1, dim)),
            full((dim, nherbs)),
            full((1, nherbs)),
            full((dim, nherbs)),
            full((1, nherbs)),
            *adj_specs,
        ],
        out_specs=pl.BlockSpec((ROW_BLOCK, nherbs), lambda i: (i, 0)),
        out_shape=jax.ShapeDtypeStruct((N, nherbs), jnp.float32),
        compiler_params=pltpu.CompilerParams(
            dimension_semantics=("arbitrary",)),
    )(s2, b2.reshape(1, dim), Wsh.T, bsh.reshape(1, nherbs),
      Whc.T, bhc.reshape(1, nherbs), *adj_args)

    return (out[:NUM_SYMPS], out[NUM_SYMPS:])


# speedup vs baseline: 1.5948x; 1.5948x over previous
import jax
import jax.numpy as jnp
from jax.experimental import pallas as pl
from jax.experimental.pallas import tpu as pltpu

N = 10000
NUM_SYMPS = 360
RB1 = 400
RB2 = 512
NUM_HERBS = 753
NHID = 64
DIM = 64


def _dot(a, b, dn=None):
    if dn is None:
        dn = (((1,), (0,)), ((), ()))
    return jax.lax.dot_general(a, b, dimension_numbers=dn,
                               precision=jax.lax.Precision.DEFAULT,
                               preferred_element_type=jnp.float32)


_DN_T = (((1,), (1,)), ((), ()))


def _s1_kernel(x_ref, w1_ref, s1_ref):
    s1_ref[...] = _dot(x_ref[...], w1_ref[...])


def _pass1_kernel(s1_ref, b1_ref, w2_ref, adj_ref, s2_ref):
    h = jnp.maximum(_dot(adj_ref[...], s1_ref[...]) + b1_ref[...], 0.0)
    s2_ref[...] = _dot(h, w2_ref[...])


def _pass2_kernel(s2_ref, b2_ref, wsh_ref, bsh_ref, whc_ref, bhc_ref,
                  adj_ref, sh_ref, ht_ref):
    h2 = _dot(adj_ref[...], s2_ref[...]) + b2_ref[...]

    @pl.when(pl.program_id(0) == 0)
    def _():
        logits_s = _dot(h2[:NUM_SYMPS], wsh_ref[...], _DN_T) + bsh_ref[...]
        sh_ref[...] = jax.nn.sigmoid(logits_s)

    logits_t = _dot(whc_ref[...], h2, _DN_T) + bhc_ref[...]
    ht_ref[...] = jax.nn.sigmoid(logits_t)


def _shift_kernel(a_ref, b_ref, out_ref):
    w = RB2 - NUM_SYMPS
    out_ref[:, :w] = a_ref[:, NUM_SYMPS:]
    out_ref[:, w:] = b_ref[:, :NUM_SYMPS]


@jax.jit
def kernel(x, adj, W1, b1, W2, b2, Wsh, bsh, Whc, bhc):
    nfeat = x.shape[1]

    s1 = pl.pallas_call(
        _s1_kernel,
        out_shape=jax.ShapeDtypeStruct((N, NHID), jnp.float32),
    )(x, W1)

    full = lambda shape: pl.BlockSpec(shape, lambda i: (0, 0))

    s2 = pl.pallas_call(
        _pass1_kernel,
        grid=(N // RB1,),
        in_specs=[
            full((N, NHID)),
            full((1, NHID)),
            full((NHID, DIM)),
            pl.BlockSpec((RB1, N), lambda i: (i, 0)),
        ],
        out_specs=pl.BlockSpec((RB1, DIM), lambda i: (i, 0)),
        out_shape=jax.ShapeDtypeStruct((N, DIM), jnp.float32),
    )(s1, b1.reshape(1, NHID), W2, adj)

    num_blocks2 = pl.cdiv(N, RB2)
    sh, ht = pl.pallas_call(
        _pass2_kernel,
        grid=(num_blocks2,),
        in_specs=[
            full((N, DIM)),
            full((1, DIM)),
            full((NUM_HERBS, DIM)),
            full((1, NUM_HERBS)),
            full((NUM_HERBS, DIM)),
            full((NUM_HERBS, 1)),
            pl.BlockSpec((RB2, N), lambda i: (i, 0)),
        ],
        out_specs=[
            pl.BlockSpec((NUM_SYMPS, NUM_HERBS), lambda i: (0, 0)),
            pl.BlockSpec((NUM_HERBS, RB2), lambda i: (0, i)),
        ],
        out_shape=[
            jax.ShapeDtypeStruct((NUM_SYMPS, NUM_HERBS), jnp.float32),
            jax.ShapeDtypeStruct((NUM_HERBS, N), jnp.float32),
        ],
    )(s2, b2.reshape(1, DIM), Wsh, bsh.reshape(1, NUM_HERBS),
      Whc, bhc.reshape(NUM_HERBS, 1), adj)

    hct = pl.pallas_call(
        _shift_kernel,
        grid=(pl.cdiv(N - NUM_SYMPS, RB2),),
        in_specs=[
            pl.BlockSpec((NUM_HERBS, RB2), lambda i: (0, i)),
            pl.BlockSpec((NUM_HERBS, RB2), lambda i: (0, i + 1)),
        ],
        out_specs=pl.BlockSpec((NUM_HERBS, RB2), lambda i: (0, i)),
        out_shape=jax.ShapeDtypeStruct((NUM_HERBS, N - NUM_SYMPS),
                                       jnp.float32),
    )(ht, ht)

    return (sh, hct.T)


# speedup vs baseline: 1.7802x; 1.1162x over previous
import jax
import jax.numpy as jnp
from jax.experimental import pallas as pl
from jax.experimental.pallas import tpu as pltpu

N = 10000
NUM_SYMPS = 360
RB1 = 400
RB2 = 512
NUM_HERBS = 753
NHID = 64
DIM = 64


def _dot(a, b, dn=None):
    if dn is None:
        dn = (((1,), (0,)), ((), ()))
    return jax.lax.dot_general(a, b, dimension_numbers=dn,
                               precision=jax.lax.Precision.DEFAULT,
                               preferred_element_type=jnp.float32)


_DN_T = (((1,), (1,)), ((), ()))


def _s1_kernel(x_ref, w1_ref, s1_ref):
    s1_ref[...] = _dot(x_ref[...], w1_ref[...])


def _pass1_kernel(s1_ref, b1_ref, w2_ref, adj_ref, s2_ref):
    h = jnp.maximum(_dot(adj_ref[...], s1_ref[...]) + b1_ref[...], 0.0)
    s2_ref[...] = _dot(h, w2_ref[...])


def _pass2_kernel(s2_ref, b2_ref, wsh_ref, bsh_ref, whc_ref, bhc_ref,
                  adj_ref, sh_ref, hct_ref, ring_ref):
    i = pl.program_id(0)
    h2 = _dot(adj_ref[...], s2_ref[...]) + b2_ref[...]

    @pl.when(i == 0)
    def _():
        logits_s = _dot(h2[:NUM_SYMPS], wsh_ref[...], _DN_T) + bsh_ref[...]
        sh_ref[...] = jax.nn.sigmoid(logits_s)

    logits_t = _dot(whc_ref[...], h2, _DN_T) + bhc_ref[...]
    cur = jax.nn.sigmoid(logits_t)
    w = RB2 - NUM_SYMPS

    @pl.when(i > 0)
    def _():
        prev = ring_ref[(i - 1) % 2]
        hct_ref[:, :w] = prev[:, NUM_SYMPS:]
        hct_ref[:, w:] = cur[:, :NUM_SYMPS]

    ring_ref[i % 2] = cur


@jax.jit
def kernel(x, adj, W1, b1, W2, b2, Wsh, bsh, Whc, bhc):
    nfeat = x.shape[1]

    s1 = pl.pallas_call(
        _s1_kernel,
        out_shape=jax.ShapeDtypeStruct((N, NHID), jnp.float32),
    )(x, W1)

    full = lambda shape: pl.BlockSpec(shape, lambda i: (0, 0))

    s2 = pl.pallas_call(
        _pass1_kernel,
        grid=(N // RB1,),
        in_specs=[
            full((N, NHID)),
            full((1, NHID)),
            full((NHID, DIM)),
            pl.BlockSpec((RB1, N), lambda i: (i, 0)),
        ],
        out_specs=pl.BlockSpec((RB1, DIM), lambda i: (i, 0)),
        out_shape=jax.ShapeDtypeStruct((N, DIM), jnp.float32),
    )(s1, b1.reshape(1, NHID), W2, adj)

    num_blocks2 = pl.cdiv(N, RB2)
    sh, hct = pl.pallas_call(
        _pass2_kernel,
        grid=(num_blocks2,),
        in_specs=[
            full((N, DIM)),
            full((1, DIM)),
            full((NUM_HERBS, DIM)),
            full((1, NUM_HERBS)),
            full((NUM_HERBS, DIM)),
            full((NUM_HERBS, 1)),
            pl.BlockSpec((RB2, N), lambda i: (i, 0)),
        ],
        out_specs=[
            pl.BlockSpec((NUM_SYMPS, NUM_HERBS), lambda i: (0, 0)),
            pl.BlockSpec((NUM_HERBS, RB2),
                         lambda i: (0, jnp.maximum(i - 1, 0))),
        ],
        out_shape=[
            jax.ShapeDtypeStruct((NUM_SYMPS, NUM_HERBS), jnp.float32),
            jax.ShapeDtypeStruct((NUM_HERBS, N - NUM_SYMPS), jnp.float32),
        ],
        scratch_shapes=[pltpu.VMEM((2, NUM_HERBS, RB2), jnp.float32)],
    )(s2, b2.reshape(1, DIM), Wsh, bsh.reshape(1, NUM_HERBS),
      Whc, bhc.reshape(NUM_HERBS, 1), adj)

    return (sh, hct.T)


# speedup vs baseline: 1.7928x; 1.0071x over previous
import jax
import jax.numpy as jnp
from jax.experimental import pallas as pl
from jax.experimental.pallas import tpu as pltpu

N = 10000
NUM_SYMPS = 360
RB = 384
NUM_HERBS = 753
NHID = 64
DIM = 64
NB = 27


def _dot(a, b, dn=None):
    if dn is None:
        dn = (((1,), (0,)), ((), ()))
    return jax.lax.dot_general(a, b, dimension_numbers=dn,
                               precision=jax.lax.Precision.DEFAULT,
                               preferred_element_type=jnp.float32)


_DN_T = (((1,), (1,)), ((), ()))


def _s1_kernel(x_ref, w1_ref, s1_ref):
    s1_ref[...] = _dot(x_ref[...], w1_ref[...])


def _mega_kernel(s1_ref, b1_ref, w2_ref, b2_ref, wsh_ref, bsh_ref,
                 whc_ref, bhc_ref, adj_ref, sh_ref, hct_ref,
                 s2_ref, ring_ref):
    i = pl.program_id(0)

    @pl.when(i < NB)
    def _():
        h = jnp.maximum(_dot(adj_ref[...], s1_ref[...]) + b1_ref[...], 0.0)
        s2_ref[pl.ds(i * RB, RB), :] = _dot(h, w2_ref[...])

    @pl.when(i >= NB)
    def _():
        k = i - NB
        h2 = _dot(adj_ref[...], s2_ref[:N, :]) + b2_ref[...]

        @pl.when(k == 0)
        def _():
            logits_s = (_dot(h2[:NUM_SYMPS], wsh_ref[...], _DN_T)
                        + bsh_ref[...])
            sh_ref[...] = jax.nn.sigmoid(logits_s)

        logits_t = _dot(whc_ref[...], h2, _DN_T) + bhc_ref[...]
        cur = jax.nn.sigmoid(logits_t)
        w = RB - NUM_SYMPS

        @pl.when(k > 0)
        def _():
            prev = ring_ref[(k - 1) % 2]
            hct_ref[:, :w] = prev[:, NUM_SYMPS:]
            hct_ref[:, w:] = cur[:, :NUM_SYMPS]

        ring_ref[k % 2] = cur


@jax.jit
def kernel(x, adj, W1, b1, W2, b2, Wsh, bsh, Whc, bhc):
    s1 = pl.pallas_call(
        _s1_kernel,
        out_shape=jax.ShapeDtypeStruct((N, NHID), jnp.float32),
    )(x, W1)

    full = lambda shape: pl.BlockSpec(shape, lambda i: (0, 0))

    sh, hct = pl.pallas_call(
        _mega_kernel,
        grid=(2 * NB,),
        in_specs=[
            full((N, NHID)),
            full((1, NHID)),
            full((NHID, DIM)),
            full((1, DIM)),
            full((NUM_HERBS, DIM)),
            full((1, NUM_HERBS)),
            full((NUM_HERBS, DIM)),
            full((NUM_HERBS, 1)),
            pl.BlockSpec((RB, N),
                         lambda i: (jnp.where(i < NB, i, i - NB), 0)),
        ],
        out_specs=[
            pl.BlockSpec((NUM_SYMPS, NUM_HERBS), lambda i: (0, 0)),
            pl.BlockSpec(
                (NUM_HERBS, RB),
                lambda i: (0, jnp.clip(i - NB - 1, 0, (N - NUM_SYMPS - 1) // RB))),
        ],
        out_shape=[
            jax.ShapeDtypeStruct((NUM_SYMPS, NUM_HERBS), jnp.float32),
            jax.ShapeDtypeStruct((NUM_HERBS, N - NUM_SYMPS), jnp.float32),
        ],
        scratch_shapes=[
            pltpu.VMEM((NB * RB, DIM), jnp.float32),
            pltpu.VMEM((2, NUM_HERBS, RB), jnp.float32),
        ],
    )(s1, b1.reshape(1, NHID), W2, b2.reshape(1, DIM),
      Wsh, bsh.reshape(1, NUM_HERBS), Whc, bhc.reshape(NUM_HERBS, 1), adj)

    return (sh, hct.T)


# speedup vs baseline: 1.8459x; 1.0296x over previous
import jax
import jax.numpy as jnp
from jax.experimental import pallas as pl
from jax.experimental.pallas import tpu as pltpu

N = 10000
NUM_SYMPS = 360
RB = 512
NUM_HERBS = 753
NHID = 64
DIM = 64
NB = 20


def _dot(a, b, dn=None):
    if dn is None:
        dn = (((1,), (0,)), ((), ()))
    return jax.lax.dot_general(a, b, dimension_numbers=dn,
                               precision=jax.lax.Precision.DEFAULT,
                               preferred_element_type=jnp.float32)


_DN_T = (((1,), (1,)), ((), ()))


def _s1_kernel(x_ref, w1_ref, s1_ref):
    s1_ref[...] = _dot(x_ref[...], w1_ref[...])


def _mega_kernel(s1_ref, b1_ref, w2_ref, b2_ref, wsh_ref, bsh_ref,
                 whc_ref, bhc_ref, adj_ref, sh_ref, hct_ref,
                 s2_ref, ring_ref):
    i = pl.program_id(0)

    @pl.when(i < NB)
    def _():
        h = jnp.maximum(_dot(adj_ref[...], s1_ref[...]) + b1_ref[...], 0.0)
        s2_ref[pl.ds(i * RB, RB), :] = _dot(h, w2_ref[...])

    @pl.when(i >= NB)
    def _():
        k = i - NB
        h2 = _dot(adj_ref[...], s2_ref[:N, :]) + b2_ref[...]

        @pl.when(k == 0)
        def _():
            logits_s = (_dot(h2[:NUM_SYMPS], wsh_ref[...], _DN_T)
                        + bsh_ref[...])
            sh_ref[...] = jax.nn.sigmoid(logits_s)

        logits_t = _dot(whc_ref[...], h2, _DN_T) + bhc_ref[...]
        cur = jax.nn.sigmoid(logits_t)
        w = RB - NUM_SYMPS

        @pl.when(k > 0)
        def _():
            prev = ring_ref[(k - 1) % 2]
            hct_ref[:, :w] = prev[:, NUM_SYMPS:]
            hct_ref[:, w:] = cur[:, :NUM_SYMPS]

        ring_ref[k % 2] = cur


@jax.jit
def kernel(x, adj, W1, b1, W2, b2, Wsh, bsh, Whc, bhc):
    s1 = pl.pallas_call(
        _s1_kernel,
        out_shape=jax.ShapeDtypeStruct((N, NHID), jnp.float32),
    )(x, W1)

    full = lambda shape: pl.BlockSpec(shape, lambda i: (0, 0))

    sh, hct = pl.pallas_call(
        _mega_kernel,
        grid=(2 * NB,),
        in_specs=[
            full((N, NHID)),
            full((1, NHID)),
            full((NHID, DIM)),
            full((1, DIM)),
            full((NUM_HERBS, DIM)),
            full((1, NUM_HERBS)),
            full((NUM_HERBS, DIM)),
            full((NUM_HERBS, 1)),
            pl.BlockSpec((RB, N),
                         lambda i: (jnp.where(i < NB, i, i - NB), 0)),
        ],
        out_specs=[
            pl.BlockSpec((NUM_SYMPS, NUM_HERBS), lambda i: (0, 0)),
            pl.BlockSpec(
                (NUM_HERBS, RB),
                lambda i: (0, jnp.clip(i - NB - 1, 0, (N - NUM_SYMPS - 1) // RB))),
        ],
        out_shape=[
            jax.ShapeDtypeStruct((NUM_SYMPS, NUM_HERBS), jnp.float32),
            jax.ShapeDtypeStruct((NUM_HERBS, N - NUM_SYMPS), jnp.float32),
        ],
        scratch_shapes=[
            pltpu.VMEM((NB * RB, DIM), jnp.float32),
            pltpu.VMEM((2, NUM_HERBS, RB), jnp.float32),
        ],
        compiler_params=pltpu.CompilerParams(
            vmem_limit_bytes=62914560),
    )(s1, b1.reshape(1, NHID), W2, b2.reshape(1, DIM),
      Wsh, bsh.reshape(1, NUM_HERBS), Whc, bhc.reshape(NUM_HERBS, 1), adj)

    return (sh, hct.T)
